# baseline (speedup 1.0000x reference)
import functools

import jax
import jax.numpy as jnp
from jax import lax
from jax.experimental import pallas as pl
from jax.experimental.pallas import tpu as pltpu
from jax.experimental.pallas import tpu_sc as plsc

N_U = 2500
N_I = 7500
N = N_U + N_I
D = 128
E_A = 320000
E_S = 80000
K = 96
NC = 2
NS = 16
NW = NC * NS
CA_W = 112
CS_W = 32
WIN = 8
NWIN_A = CA_W // WIN
NWIN_S = CS_W // WIN
RA0, RA_LAST = 624, N - 15 * 624
RS0, RS_LAST = 152, N_U - 15 * 152

_mesh = plsc.VectorSubcoreMesh(core_axis_name="c", subcore_axis_name="s")


@functools.partial(
    pl.kernel,
    out_type=(
        jax.ShapeDtypeStruct((NC, N, D), jnp.float32),
        jax.ShapeDtypeStruct((NC, N_U, D), jnp.float32),
    ),
    mesh=_mesh,
    scratch_types=[
        pltpu.VMEM_SHARED((N, D), jnp.float32),
        pltpu.VMEM_SHARED((N_U, D), jnp.float32),
        pltpu.VMEM((WIN, K), jnp.int32),
        pltpu.VMEM((WIN, K), jnp.int32),
        pltpu.VMEM((WIN, K), jnp.float32),
        [pltpu.VMEM((K, D), jnp.float32)] * 2,
        [pltpu.SemaphoreType.DMA] * 2,
    ],
)
def _sc_spmm(x_hbm, sx_hbm, srca_hbm, dsta_hbm, va_hbm, srcs_hbm, dsts_hbm,
             vs_hbm, pa_hbm, ps_hbm, acc_a, acc_s, wsrc, wdst, wval, rg,
             semg):
    cid = lax.axis_index("c")
    sid = lax.axis_index("s")
    wid = sid * NC + cid

    zero = jnp.zeros((16,), jnp.float32)

    def _zrow(k, _):
        for j in range(D // 16):
            rg[0][k, pl.ds(j * 16, 16)] = zero
        return 0

    lax.fori_loop(0, K, _zrow, 0)

    def _fill(dst, base, n):
        full, rem = n // K, n % K
        for r in range(full):
            pltpu.sync_copy(rg[0].at[pl.ds(0, K)],
                            dst.at[pl.ds(base + r * K, K)])
        if rem:
            pltpu.sync_copy(rg[0].at[pl.ds(0, rem)],
                            dst.at[pl.ds(base + full * K, rem)])

    @pl.when(sid < 15)
    def _():
        _fill(acc_a, sid * RA0, RA0)
        _fill(acc_s, sid * RS0, RS0)

    @pl.when(sid == 15)
    def _():
        _fill(acc_a, 15 * RA0, RA_LAST)
        _fill(acc_s, 15 * RS0, RS_LAST)

    plsc.subcore_barrier()

    def _run_phase(nwin, srch, dsth, valh, x_ref, acc):
        def _win(w, _):
            pltpu.sync_copy(srch.at[wid, w], wsrc)
            pltpu.sync_copy(dsth.at[wid, w], wdst)
            pltpu.sync_copy(valh.at[wid, w], wval)
            for c in range(WIN):
                b = c % 2
                pltpu.async_copy(x_ref.at[wsrc.at[c]], rg[b],
                                 semg[b]).wait()

                def body(g, _b):
                    vv = wval[c, pl.ds(g * 16, 16)]
                    for e in range(16):
                        v = vv[e]
                        r = g * 16 + e
                        for j in range(D // 16):
                            rg[b][r, pl.ds(j * 16, 16)] = (
                                rg[b][r, pl.ds(j * 16, 16)] * v)
                    return 0

                lax.fori_loop(0, K // 16, body, 0)
                pltpu.sync_copy(rg[b], acc.at[wdst.at[c]], add=True)
            return 0

        lax.fori_loop(0, nwin, _win, 0)

    _run_phase(NWIN_A, srca_hbm, dsta_hbm, va_hbm, x_hbm, acc_a)
    _run_phase(NWIN_S, srcs_hbm, dsts_hbm, vs_hbm, sx_hbm, acc_s)

    plsc.subcore_barrier()

    @pl.when(sid < 15)
    def _():
        pltpu.sync_copy(acc_a.at[pl.ds(sid * RA0, RA0)],
                        pa_hbm.at[cid, pl.ds(sid * RA0, RA0)])
        pltpu.sync_copy(acc_s.at[pl.ds(sid * RS0, RS0)],
                        ps_hbm.at[cid, pl.ds(sid * RS0, RS0)])

    @pl.when(sid == 15)
    def _():
        pltpu.sync_copy(acc_a.at[pl.ds(15 * RA0, RA_LAST)],
                        pa_hbm.at[cid, pl.ds(15 * RA0, RA_LAST)])
        pltpu.sync_copy(acc_s.at[pl.ds(15 * RS0, RS_LAST)],
                        ps_hbm.at[cid, pl.ds(15 * RS0, RS_LAST)])


def _mish(x):
    sp = jnp.maximum(x, 0.0) + jnp.log(1.0 + jnp.exp(-jnp.abs(x)))
    return x * jnp.tanh(sp)


def _fusion(u, s, f1w, f1b, f2w, f2b, f3w, f3b):
    c = jnp.concatenate([u, s, u * s], axis=1)
    t1 = _mish(jnp.dot(c, f1w, preferred_element_type=jnp.float32) + f1b)
    t2 = _mish(jnp.dot(t1, f2w, preferred_element_type=jnp.float32) + f2b)
    t3 = jnp.dot(t2, f3w, preferred_element_type=jnp.float32) + f3b
    return t3 / jnp.sqrt(jnp.sum(t3 * t3))


def _tc1_body(pa, ps, f1w, f1b, f2w, f2b, f3w, f3b, ego_out):
    a = pa[0] + pa[1]
    s = ps[0] + ps[1]
    u = a[:N_U]
    ego_out[pl.ds(0, N_U), :] = _fusion(u, s, f1w[...], f1b[...], f2w[...],
                                        f2b[...], f3w[...], f3b[...])
    ego_out[pl.ds(N_U, N_I), :] = a[N_U:]


def _tc2_body(qa, qs, ue, ie, ego1, f1w, f1b, f2w, f2b, f3w, f3b,
              user_out, item_out):
    a = qa[0] + qa[1]
    s = qs[0] + qs[1]
    t3n = _fusion(a[:N_U], s, f1w[...], f1b[...], f2w[...], f2b[...],
                  f3w[...], f3b[...])
    user_out[...] = (ue[...] + ego1[pl.ds(0, N_U), :] + t3n) * (1.0 / 3.0)
    item_out[...] = (ie[...] + ego1[pl.ds(N_U, N_I), :] + a[N_U:]) * (1.0 / 3.0)


_tc1 = pl.pallas_call(
    _tc1_body,
    out_shape=jax.ShapeDtypeStruct((N, D), jnp.float32),
)

_tc2 = pl.pallas_call(
    _tc2_body,
    out_shape=(
        jax.ShapeDtypeStruct((N_U, D), jnp.float32),
        jax.ShapeDtypeStruct((N_I, D), jnp.float32),
    ),
)


def _pad_edges(edge_index, vals, chunks_w):
    e = edge_index.shape[1]
    per_w = e // NW
    pad = chunks_w * K - per_w
    src = edge_index[0].reshape(NW, per_w)
    dst = edge_index[1].reshape(NW, per_w)
    v = vals.reshape(NW, per_w)
    zi = jnp.zeros((NW, pad), jnp.int32)
    di = jnp.broadcast_to(jnp.arange(pad, dtype=jnp.int32)[None, :] % N_U,
                          (NW, pad))
    zf = jnp.zeros((NW, pad), jnp.float32)
    nwin = chunks_w // WIN
    src = jnp.concatenate([src, zi], axis=1).reshape(NW, nwin, WIN, K)
    dst = jnp.concatenate([dst, di], axis=1).reshape(NW, nwin, WIN, K)
    v = jnp.concatenate([v, zf], axis=1).reshape(NW, nwin, WIN, K)
    return src, dst, v


def kernel(user_emb, item_emb, a_vals, s_vals, fc1_w, fc1_b, fc2_w, fc2_b,
           fc3_w, fc3_b, edge_index_a, edge_index_s):
    x0 = jnp.concatenate([user_emb, item_emb], axis=0)
    src_a, dst_a, va = _pad_edges(edge_index_a, a_vals, CA_W)
    src_s, dst_s, vs = _pad_edges(edge_index_s, s_vals, CS_W)

    pa, ps = _sc_spmm(x0, user_emb, src_a, dst_a, va, src_s, dst_s, vs)
    ego1 = _tc1(pa, ps, fc1_w, fc1_b, fc2_w, fc2_b, fc3_w, fc3_b)
    qa, qs = _sc_spmm(ego1, ego1[:N_U], src_a, dst_a, va, src_s, dst_s, vs)
    user_out, item_out = _tc2(qa, qs, user_emb, item_emb, ego1, fc1_w, fc1_b,
                              fc2_w, fc2_b, fc3_w, fc3_b)
    return user_out, item_out


# speedup vs baseline: 1.5005x; 1.5005x over previous
import functools

import jax
import jax.numpy as jnp
from jax import lax
from jax.experimental import pallas as pl
from jax.experimental.pallas import tpu as pltpu
from jax.experimental.pallas import tpu_sc as plsc

N_U = 2500
N_I = 7500
N = N_U + N_I
D = 128
E_A = 320000
E_S = 80000
K = 128
NC = 2
NS = 16
NW = NC * NS
CA_W = 80
CS_W = 24
WIN = 8
NWIN_A = CA_W // WIN
NWIN_S = CS_W // WIN
RA0, RA_LAST = 624, N - 15 * 624
RS0, RS_LAST = 152, N_U - 15 * 152

_mesh = plsc.VectorSubcoreMesh(core_axis_name="c", subcore_axis_name="s")


@functools.partial(
    pl.kernel,
    out_type=(
        jax.ShapeDtypeStruct((NC, N, D), jnp.float32),
        jax.ShapeDtypeStruct((NC, N_U, D), jnp.float32),
    ),
    mesh=_mesh,
    scratch_types=[
        pltpu.VMEM_SHARED((N, D), jnp.float32),
        pltpu.VMEM_SHARED((N_U, D), jnp.float32),
        pltpu.VMEM((WIN, K), jnp.int32),
        pltpu.VMEM((WIN, K), jnp.int32),
        pltpu.VMEM((WIN, K), jnp.float32),
        pltpu.VMEM((K,), jnp.int32),
        pltpu.VMEM((K,), jnp.int32),
        pltpu.VMEM((K, D), jnp.float32),
        pltpu.SemaphoreType.DMA,
    ],
)
def _sc_spmm(x_hbm, sx_hbm, srca_hbm, dsta_hbm, va_hbm, srcs_hbm, dsts_hbm,
             vs_hbm, pa_hbm, ps_hbm, acc_a, acc_s, wsrc, wdst, wval, srcv,
             dstv, rg, semg):
    cid = lax.axis_index("c")
    sid = lax.axis_index("s")
    wid = sid * NC + cid

    zero = jnp.zeros((16,), jnp.float32)

    def _zrow(k, _):
        for j in range(D // 16):
            rg[k, pl.ds(j * 16, 16)] = zero
        return 0

    lax.fori_loop(0, K, _zrow, 0)

    def _fill(dst, base, n):
        full, rem = n // K, n % K
        for r in range(full):
            pltpu.sync_copy(rg.at[pl.ds(0, K)],
                            dst.at[pl.ds(base + r * K, K)])
        if rem:
            pltpu.sync_copy(rg.at[pl.ds(0, rem)],
                            dst.at[pl.ds(base + full * K, rem)])

    @pl.when(sid < 15)
    def _():
        _fill(acc_a, sid * RA0, RA0)
        _fill(acc_s, sid * RS0, RS0)

    @pl.when(sid == 15)
    def _():
        _fill(acc_a, 15 * RA0, RA_LAST)
        _fill(acc_s, 15 * RS0, RS_LAST)

    plsc.subcore_barrier()

    def _run_phase(nwin, srch, dsth, valh, x_ref, acc):
        def _win(w, _):
            pltpu.sync_copy(srch.at[wid, w], wsrc)
            pltpu.sync_copy(dsth.at[wid, w], wdst)
            pltpu.sync_copy(valh.at[wid, w], wval)
            for c in range(WIN):
                for g in range(K // 16):
                    srcv[pl.ds(g * 16, 16)] = wsrc[c, pl.ds(g * 16, 16)]
                    dstv[pl.ds(g * 16, 16)] = wdst[c, pl.ds(g * 16, 16)]
                pltpu.async_copy(x_ref.at[srcv], rg, semg).wait()

                def body(g, _b):
                    vv = wval[c, pl.ds(g * 16, 16)]
                    for e in range(16):
                        v = vv[e]
                        r = g * 16 + e
                        for j in range(D // 16):
                            rg[r, pl.ds(j * 16, 16)] = (
                                rg[r, pl.ds(j * 16, 16)] * v)
                    return 0

                lax.fori_loop(0, K // 16, body, 0)
                pltpu.sync_copy(rg, acc.at[dstv], add=True)
            return 0

        lax.fori_loop(0, nwin, _win, 0)

    _run_phase(NWIN_A, srca_hbm, dsta_hbm, va_hbm, x_hbm, acc_a)
    _run_phase(NWIN_S, srcs_hbm, dsts_hbm, vs_hbm, sx_hbm, acc_s)

    plsc.subcore_barrier()

    @pl.when(sid < 15)
    def _():
        pltpu.sync_copy(acc_a.at[pl.ds(sid * RA0, RA0)],
                        pa_hbm.at[cid, pl.ds(sid * RA0, RA0)])
        pltpu.sync_copy(acc_s.at[pl.ds(sid * RS0, RS0)],
                        ps_hbm.at[cid, pl.ds(sid * RS0, RS0)])

    @pl.when(sid == 15)
    def _():
        pltpu.sync_copy(acc_a.at[pl.ds(15 * RA0, RA_LAST)],
                        pa_hbm.at[cid, pl.ds(15 * RA0, RA_LAST)])
        pltpu.sync_copy(acc_s.at[pl.ds(15 * RS0, RS_LAST)],
                        ps_hbm.at[cid, pl.ds(15 * RS0, RS_LAST)])


def _mish(x):
    sp = jnp.maximum(x, 0.0) + jnp.log(1.0 + jnp.exp(-jnp.abs(x)))
    return x * jnp.tanh(sp)


def _fusion(u, s, f1w, f1b, f2w, f2b, f3w, f3b):
    c = jnp.concatenate([u, s, u * s], axis=1)
    t1 = _mish(jnp.dot(c, f1w, preferred_element_type=jnp.float32) + f1b)
    t2 = _mish(jnp.dot(t1, f2w, preferred_element_type=jnp.float32) + f2b)
    t3 = jnp.dot(t2, f3w, preferred_element_type=jnp.float32) + f3b
    return t3 / jnp.sqrt(jnp.sum(t3 * t3))


def _tc1_body(pa, ps, f1w, f1b, f2w, f2b, f3w, f3b, ego_out):
    a = pa[0] + pa[1]
    s = ps[0] + ps[1]
    u = a[:N_U]
    ego_out[pl.ds(0, N_U), :] = _fusion(u, s, f1w[...], f1b[...], f2w[...],
                                        f2b[...], f3w[...], f3b[...])
    ego_out[pl.ds(N_U, N_I), :] = a[N_U:]


def _tc2_body(qa, qs, ue, ie, ego1, f1w, f1b, f2w, f2b, f3w, f3b,
              user_out, item_out):
    a = qa[0] + qa[1]
    s = qs[0] + qs[1]
    t3n = _fusion(a[:N_U], s, f1w[...], f1b[...], f2w[...], f2b[...],
                  f3w[...], f3b[...])
    user_out[...] = (ue[...] + ego1[pl.ds(0, N_U), :] + t3n) * (1.0 / 3.0)
    item_out[...] = (ie[...] + ego1[pl.ds(N_U, N_I), :] + a[N_U:]) * (1.0 / 3.0)


_tc1 = pl.pallas_call(
    _tc1_body,
    out_shape=jax.ShapeDtypeStruct((N, D), jnp.float32),
)

_tc2 = pl.pallas_call(
    _tc2_body,
    out_shape=(
        jax.ShapeDtypeStruct((N_U, D), jnp.float32),
        jax.ShapeDtypeStruct((N_I, D), jnp.float32),
    ),
)


def _pad_edges(edge_index, vals, chunks_w):
    e = edge_index.shape[1]
    per_w = e // NW
    pad = chunks_w * K - per_w
    src = edge_index[0].reshape(NW, per_w)
    dst = edge_index[1].reshape(NW, per_w)
    v = vals.reshape(NW, per_w)
    zi = jnp.zeros((NW, pad), jnp.int32)
    di = jnp.broadcast_to(jnp.arange(pad, dtype=jnp.int32)[None, :] % N_U,
                          (NW, pad))
    zf = jnp.zeros((NW, pad), jnp.float32)
    nwin = chunks_w // WIN
    src = jnp.concatenate([src, zi], axis=1).reshape(NW, nwin, WIN, K)
    dst = jnp.concatenate([dst, di], axis=1).reshape(NW, nwin, WIN, K)
    v = jnp.concatenate([v, zf], axis=1).reshape(NW, nwin, WIN, K)
    return src, dst, v


def kernel(user_emb, item_emb, a_vals, s_vals, fc1_w, fc1_b, fc2_w, fc2_b,
           fc3_w, fc3_b, edge_index_a, edge_index_s):
    x0 = jnp.concatenate([user_emb, item_emb], axis=0)
    src_a, dst_a, va = _pad_edges(edge_index_a, a_vals, CA_W)
    src_s, dst_s, vs = _pad_edges(edge_index_s, s_vals, CS_W)

    pa, ps = _sc_spmm(x0, user_emb, src_a, dst_a, va, src_s, dst_s, vs)
    ego1 = _tc1(pa, ps, fc1_w, fc1_b, fc2_w, fc2_b, fc3_w, fc3_b)
    qa, qs = _sc_spmm(ego1, ego1[:N_U], src_a, dst_a, va, src_s, dst_s, vs)
    user_out, item_out = _tc2(qa, qs, user_emb, item_emb, ego1, fc1_w, fc1_b,
                              fc2_w, fc2_b, fc3_w, fc3_b)
    return user_out, item_out


# speedup vs baseline: 4.7867x; 3.1901x over previous
import functools

import jax
import jax.numpy as jnp
from jax import lax
from jax.experimental import pallas as pl
from jax.experimental.pallas import tpu as pltpu
from jax.experimental.pallas import tpu_sc as plsc

N_U = 2500
N_I = 7500
N = N_U + N_I
D = 128
E_A = 320000
E_S = 80000
K = 128
PK = 2 * K
NC = 2
NS = 16
NW = NC * NS
CA = E_A // K
CS = E_S // K
RA0, RA_LAST = 624, N - 15 * 624
RS0, RS_LAST = 152, N_U - 15 * 152

_mesh = plsc.VectorSubcoreMesh(core_axis_name="c", subcore_axis_name="s")


@functools.partial(
    pl.kernel,
    out_type=(
        jax.ShapeDtypeStruct((NC, N, D), jnp.float32),
        jax.ShapeDtypeStruct((NC, N_U, D), jnp.float32),
    ),
    mesh=_mesh,
    scratch_types=[
        pltpu.VMEM_SHARED((N, D), jnp.float32),
        pltpu.VMEM_SHARED((N_U, D), jnp.float32),
        pltpu.VMEM((PK,), jnp.int32),
        pltpu.VMEM((K,), jnp.int32),
        pltpu.VMEM((K,), jnp.int32),
        pltpu.VMEM((K,), jnp.float32),
        pltpu.VMEM((K, D), jnp.float32),
        pltpu.SemaphoreType.DMA,
    ],
)
def _sc_spmm(x_hbm, sx_hbm, pka_hbm, pks_hbm, va_hbm, vs_hbm, pa_hbm,
             ps_hbm, acc_a, acc_s, pkv, srcv, dstv, valv, rows, sem):
    cid = lax.axis_index("c")
    sid = lax.axis_index("s")
    wid = sid * NC + cid

    zero = jnp.zeros((16,), jnp.float32)

    def _zrow(k, _):
        for j in range(D // 16):
            rows[k, pl.ds(j * 16, 16)] = zero
        return 0

    lax.fori_loop(0, K, _zrow, 0)

    def _fill(dst, base, n):
        full, rem = n // K, n % K
        for r in range(full):
            pltpu.sync_copy(rows.at[pl.ds(0, K)],
                            dst.at[pl.ds(base + r * K, K)])
        if rem:
            pltpu.sync_copy(rows.at[pl.ds(0, rem)],
                            dst.at[pl.ds(base + full * K, rem)])

    @pl.when(sid < 15)
    def _():
        _fill(acc_a, sid * RA0, RA0)
        _fill(acc_s, sid * RS0, RS0)

    @pl.when(sid == 15)
    def _():
        _fill(acc_a, 15 * RA0, RA_LAST)
        _fill(acc_s, 15 * RS0, RS_LAST)

    plsc.subcore_barrier()

    def _edge_chunk(c, pk_hbm, v_hbm, x_ref, acc):
        pltpu.sync_copy(pk_hbm.at[pl.ds(c * PK, PK)], pkv)
        pltpu.sync_copy(v_hbm.at[pl.ds(c * K, K)], valv)
        for g in range(K // 16):
            srcv[pl.ds(g * 16, 16)] = pkv[pl.ds(g * 16, 16)]
            dstv[pl.ds(g * 16, 16)] = pkv[pl.ds(K + g * 16, 16)]
        pltpu.async_copy(x_ref.at[srcv], rows, sem).wait()

        def _scale(g, _):
            vv = valv[pl.ds(g * 16, 16)]
            for e in range(16):
                v = vv[e]
                r = g * 16 + e
                for j in range(D // 16):
                    rows[r, pl.ds(j * 16, 16)] = rows[r, pl.ds(j * 16, 16)] * v
            return 0

        lax.fori_loop(0, K // 16, _scale, 0)
        pltpu.sync_copy(rows, acc.at[dstv], add=True)

    na = (CA - 1 - wid) // NW + 1

    def _a_body(i, _):
        _edge_chunk(wid + i * NW, pka_hbm, va_hbm, x_hbm, acc_a)
        return 0

    lax.fori_loop(0, na, _a_body, 0)

    ns = (CS - 1 - wid) // NW + 1

    def _s_body(i, _):
        _edge_chunk(wid + i * NW, pks_hbm, vs_hbm, sx_hbm, acc_s)
        return 0

    lax.fori_loop(0, ns, _s_body, 0)

    plsc.subcore_barrier()

    @pl.when(sid < 15)
    def _():
        pltpu.sync_copy(acc_a.at[pl.ds(sid * RA0, RA0)],
                        pa_hbm.at[cid, pl.ds(sid * RA0, RA0)])
        pltpu.sync_copy(acc_s.at[pl.ds(sid * RS0, RS0)],
                        ps_hbm.at[cid, pl.ds(sid * RS0, RS0)])

    @pl.when(sid == 15)
    def _():
        pltpu.sync_copy(acc_a.at[pl.ds(15 * RA0, RA_LAST)],
                        pa_hbm.at[cid, pl.ds(15 * RA0, RA_LAST)])
        pltpu.sync_copy(acc_s.at[pl.ds(15 * RS0, RS_LAST)],
                        ps_hbm.at[cid, pl.ds(15 * RS0, RS_LAST)])


def _mish(x):
    sp = jnp.maximum(x, 0.0) + jnp.log(1.0 + jnp.exp(-jnp.abs(x)))
    return x * jnp.tanh(sp)


def _fusion(u, s, f1w, f1b, f2w, f2b, f3w, f3b):
    c = jnp.concatenate([u, s, u * s], axis=1)
    t1 = _mish(jnp.dot(c, f1w, preferred_element_type=jnp.float32) + f1b)
    t2 = _mish(jnp.dot(t1, f2w, preferred_element_type=jnp.float32) + f2b)
    t3 = jnp.dot(t2, f3w, preferred_element_type=jnp.float32) + f3b
    return t3 / jnp.sqrt(jnp.sum(t3 * t3))


def _tc1_body(pa, ps, f1w, f1b, f2w, f2b, f3w, f3b, ego_out):
    a = pa[0] + pa[1]
    s = ps[0] + ps[1]
    u = a[:N_U]
    ego_out[pl.ds(0, N_U), :] = _fusion(u, s, f1w[...], f1b[...], f2w[...],
                                        f2b[...], f3w[...], f3b[...])
    ego_out[pl.ds(N_U, N_I), :] = a[N_U:]


def _tc2_body(qa, qs, ue, ie, ego1, f1w, f1b, f2w, f2b, f3w, f3b,
              user_out, item_out):
    a = qa[0] + qa[1]
    s = qs[0] + qs[1]
    t3n = _fusion(a[:N_U], s, f1w[...], f1b[...], f2w[...], f2b[...],
                  f3w[...], f3b[...])
    user_out[...] = (ue[...] + ego1[pl.ds(0, N_U), :] + t3n) * (1.0 / 3.0)
    item_out[...] = (ie[...] + ego1[pl.ds(N_U, N_I), :] + a[N_U:]) * (1.0 / 3.0)


_tc1 = pl.pallas_call(
    _tc1_body,
    out_shape=jax.ShapeDtypeStruct((N, D), jnp.float32),
)

_tc2 = pl.pallas_call(
    _tc2_body,
    out_shape=(
        jax.ShapeDtypeStruct((N_U, D), jnp.float32),
        jax.ShapeDtypeStruct((N_I, D), jnp.float32),
    ),
)


def _pack_edges(edge_index):
    e = edge_index.shape[1]
    nchunks = e // K
    src = edge_index[0].reshape(nchunks, K)
    dst = edge_index[1].reshape(nchunks, K)
    return jnp.concatenate([src, dst], axis=1).reshape(-1)


def kernel(user_emb, item_emb, a_vals, s_vals, fc1_w, fc1_b, fc2_w, fc2_b,
           fc3_w, fc3_b, edge_index_a, edge_index_s):
    x0 = jnp.concatenate([user_emb, item_emb], axis=0)
    pka = _pack_edges(edge_index_a)
    pks = _pack_edges(edge_index_s)

    pa, ps = _sc_spmm(x0, user_emb, pka, pks, a_vals, s_vals)
    ego1 = _tc1(pa, ps, fc1_w, fc1_b, fc2_w, fc2_b, fc3_w, fc3_b)
    qa, qs = _sc_spmm(ego1, ego1[:N_U], pka, pks, a_vals, s_vals)
    user_out, item_out = _tc2(qa, qs, user_emb, item_emb, ego1, fc1_w, fc1_b,
                              fc2_w, fc2_b, fc3_w, fc3_b)
    return user_out, item_out
